# initial kernel scaffold (unmeasured)
import jax
import jax.numpy as jnp
from jax import lax
from jax.experimental import pallas as pl
from jax.experimental.pallas import tpu as pltpu


def kernel(
    x,
):
    def body(*refs):
        pass

    out_shape = jax.ShapeDtypeStruct(..., jnp.float32)
    return pl.pallas_call(body, out_shape=out_shape)(...)


# baseline (device time: 1255275 ns/iter reference)
import jax
import jax.numpy as jnp
from jax import lax
from jax.experimental import pallas as pl
from jax.experimental.pallas import tpu as pltpu

N_DEV = 4
N_SUB = 2


def kernel(x):
    x16 = x.astype(jnp.bfloat16)
    m_per, n = x16.shape
    sub_m = m_per // N_SUB

    def body(x_ref, out_ref, comm_ref, init_sems, copy_sems, send_sems, recv_sems):
        my = lax.axis_index("i")
        left = (my + N_DEV - 1) % N_DEV
        right = (my + 1) % N_DEV

        barrier_sem = pltpu.get_barrier_semaphore()
        for nbr in (left, right):
            pl.semaphore_signal(
                barrier_sem, inc=1,
                device_id=(nbr,), device_id_type=pl.DeviceIdType.MESH,
            )
        pl.semaphore_wait(barrier_sem, 2)

        cp_out = pltpu.make_async_copy(
            x_ref, out_ref.at[pl.ds(my * m_per, m_per)], init_sems.at[0]
        )
        cp_out.start()
        for s in range(N_SUB):
            cp = pltpu.make_async_copy(
                x_ref.at[pl.ds(s * sub_m, sub_m)], comm_ref.at[s], init_sems.at[1 + s]
            )
            cp.start()
            cp.wait()

        for h in range(N_DEV - 1):
            origin = (my + N_DEV - h - 1) % N_DEV
            for s in range(N_SUB):
                rdma = pltpu.make_async_remote_copy(
                    src_ref=comm_ref.at[s],
                    dst_ref=comm_ref.at[s],
                    send_sem=send_sems.at[h, s],
                    recv_sem=recv_sems.at[h, s],
                    device_id=(right,),
                    device_id_type=pl.DeviceIdType.MESH,
                )
                rdma.start()
                rdma.wait()
                cp = pltpu.make_async_copy(
                    comm_ref.at[s],
                    out_ref.at[pl.ds(origin * m_per + s * sub_m, sub_m)],
                    copy_sems.at[h, s],
                )
                cp.start()
                cp.wait()

        cp_out.wait()

    return pl.pallas_call(
        body,
        out_shape=jax.ShapeDtypeStruct((N_DEV * m_per, n), jnp.bfloat16),
        in_specs=[pl.BlockSpec(memory_space=pl.ANY)],
        out_specs=pl.BlockSpec(memory_space=pl.ANY),
        scratch_shapes=[
            pltpu.VMEM((N_SUB, sub_m, n), jnp.bfloat16),
            pltpu.SemaphoreType.DMA((1 + N_SUB,)),
            pltpu.SemaphoreType.DMA((N_DEV - 1, N_SUB)),
            pltpu.SemaphoreType.DMA((N_DEV - 1, N_SUB)),
            pltpu.SemaphoreType.DMA((N_DEV - 1, N_SUB)),
        ],
        compiler_params=pltpu.CompilerParams(collective_id=0),
    )(x16)


# device time: 1140016 ns/iter; 1.1011x vs baseline; 1.1011x over previous
import jax
import jax.numpy as jnp
from jax import lax
from jax.experimental import pallas as pl
from jax.experimental.pallas import tpu as pltpu

N_DEV = 4
S = 16
H = S // 2


def kernel(x):
    x16 = x.astype(jnp.bfloat16)
    m_per, n = x16.shape
    sub = m_per // S

    def body(
        x_ref, out_ref,
        sendbuf, recvL, recvR, recvE, recvF,
        own_sem, load_sems, sendA, sendB, recvC, recvD,
        fwdsR, fwdsL, recvEs, recvFs,
        copyCs, copyDs, copyEs, copyFs,
        credA, credB, credFR, credFL,
    ):
        my = lax.axis_index("i")
        left = (my + N_DEV - 1) % N_DEV
        right = (my + 1) % N_DEV
        opp = (my + 2) % N_DEV
        base_my = my * m_per
        base_l = left * m_per
        base_r = right * m_per
        base_o = opp * m_per

        barrier_sem = pltpu.get_barrier_semaphore()
        for nbr in (left, right):
            pl.semaphore_signal(
                barrier_sem, inc=1,
                device_id=(nbr,), device_id_type=pl.DeviceIdType.MESH,
            )
        pl.semaphore_wait(barrier_sem, 2)

        own_cp = pltpu.make_async_copy(
            x_ref, out_ref.at[pl.ds(base_my, m_per)], own_sem
        )
        own_cp.start()

        loads, A, B, fR, fL = {}, {}, {}, {}, {}
        copyC, copyD, copyE, copyF = {}, {}, {}, {}

        def start_load(k):
            d = pltpu.make_async_copy(
                x_ref.at[pl.ds(k * sub, sub)], sendbuf.at[k % 2],
                load_sems.at[k % 2],
            )
            loads[k] = d
            d.start()

        start_load(0)
        start_load(1)

        for k in range(S):
            if 2 <= k <= H + 1:
                jj = k - 2
                copyE[jj].wait()
                if jj <= H - 3:
                    pl.semaphore_signal(
                        credFR, inc=1,
                        device_id=(left,), device_id_type=pl.DeviceIdType.MESH,
                    )
            if k >= H + 2:
                jj = k - (H + 2)
                copyF[jj].wait()
                if jj <= H - 3:
                    pl.semaphore_signal(
                        credFL, inc=1,
                        device_id=(right,), device_id_type=pl.DeviceIdType.MESH,
                    )

            loads[k].wait()
            if k >= 4:
                pl.semaphore_wait(credA, 1)
                pl.semaphore_wait(credB, 1)
            A[k] = pltpu.make_async_remote_copy(
                src_ref=sendbuf.at[k % 2], dst_ref=recvL.at[k % 4],
                send_sem=sendA.at[k % 2], recv_sem=recvC.at[k % 2],
                device_id=(right,), device_id_type=pl.DeviceIdType.MESH,
            )
            A[k].start()
            B[k] = pltpu.make_async_remote_copy(
                src_ref=sendbuf.at[k % 2], dst_ref=recvR.at[k % 4],
                send_sem=sendB.at[k % 2], recv_sem=recvD.at[k % 2],
                device_id=(left,), device_id_type=pl.DeviceIdType.MESH,
            )
            B[k].start()

            A[k].wait_recv()
            copyC[k] = pltpu.make_async_copy(
                recvL.at[k % 4],
                out_ref.at[pl.ds(base_l + k * sub, sub)],
                copyCs.at[k % 2],
            )
            copyC[k].start()
            if k < H:
                if k >= 2:
                    pl.semaphore_wait(credFR, 1)
                fR[k] = pltpu.make_async_remote_copy(
                    src_ref=recvL.at[k % 4], dst_ref=recvE.at[k % 2],
                    send_sem=fwdsR.at[k % 2], recv_sem=recvEs.at[k % 2],
                    device_id=(right,), device_id_type=pl.DeviceIdType.MESH,
                )
                fR[k].start()

            B[k].wait_recv()
            copyD[k] = pltpu.make_async_copy(
                recvR.at[k % 4],
                out_ref.at[pl.ds(base_r + k * sub, sub)],
                copyDs.at[k % 2],
            )
            copyD[k].start()
            if k >= H:
                j = k - H
                if j >= 2:
                    pl.semaphore_wait(credFL, 1)
                fL[j] = pltpu.make_async_remote_copy(
                    src_ref=recvR.at[k % 4], dst_ref=recvF.at[j % 2],
                    send_sem=fwdsL.at[j % 2], recv_sem=recvFs.at[j % 2],
                    device_id=(left,), device_id_type=pl.DeviceIdType.MESH,
                )
                fL[j].start()

            if k >= 1:
                kk = k - 1
                copyC[kk].wait()
                if kk < H:
                    fR[kk].wait_send()
                if kk <= S - 5:
                    pl.semaphore_signal(
                        credA, inc=1,
                        device_id=(left,), device_id_type=pl.DeviceIdType.MESH,
                    )
                copyD[kk].wait()
                if kk >= H:
                    fL[kk - H].wait_send()
                if kk <= S - 5:
                    pl.semaphore_signal(
                        credB, inc=1,
                        device_id=(right,), device_id_type=pl.DeviceIdType.MESH,
                    )

            if 1 <= k <= H:
                j = k - 1
                erecv = pltpu.make_async_remote_copy(
                    src_ref=recvL.at[j % 4], dst_ref=recvE.at[j % 2],
                    send_sem=fwdsR.at[j % 2], recv_sem=recvEs.at[j % 2],
                    device_id=(right,), device_id_type=pl.DeviceIdType.MESH,
                )
                erecv.wait_recv()
                copyE[j] = pltpu.make_async_copy(
                    recvE.at[j % 2],
                    out_ref.at[pl.ds(base_o + j * sub, sub)],
                    copyEs.at[j % 2],
                )
                copyE[j].start()

            if k >= H + 1:
                j = k - (H + 1)
                frecv = pltpu.make_async_remote_copy(
                    src_ref=recvR.at[j % 4], dst_ref=recvF.at[j % 2],
                    send_sem=fwdsL.at[j % 2], recv_sem=recvFs.at[j % 2],
                    device_id=(left,), device_id_type=pl.DeviceIdType.MESH,
                )
                frecv.wait_recv()
                copyF[j] = pltpu.make_async_copy(
                    recvF.at[j % 2],
                    out_ref.at[pl.ds(base_o + (H + j) * sub, sub)],
                    copyFs.at[j % 2],
                )
                copyF[j].start()

            if k + 2 < S:
                A[k].wait_send()
                B[k].wait_send()
                start_load(k + 2)

        for k in (S - 2, S - 1):
            A[k].wait_send()
            B[k].wait_send()
        copyC[S - 1].wait()
        copyD[S - 1].wait()
        fL[H - 1].wait_send()
        copyF[H - 2].wait()
        j = H - 1
        frecv = pltpu.make_async_remote_copy(
            src_ref=recvR.at[j % 4], dst_ref=recvF.at[j % 2],
            send_sem=fwdsL.at[j % 2], recv_sem=recvFs.at[j % 2],
            device_id=(left,), device_id_type=pl.DeviceIdType.MESH,
        )
        frecv.wait_recv()
        lastF = pltpu.make_async_copy(
            recvF.at[j % 2],
            out_ref.at[pl.ds(base_o + (H + j) * sub, sub)],
            copyFs.at[j % 2],
        )
        lastF.start()
        lastF.wait()
        own_cp.wait()

    return pl.pallas_call(
        body,
        out_shape=jax.ShapeDtypeStruct((N_DEV * m_per, n), jnp.bfloat16),
        in_specs=[pl.BlockSpec(memory_space=pl.ANY)],
        out_specs=pl.BlockSpec(memory_space=pl.ANY),
        scratch_shapes=[
            pltpu.VMEM((2, sub, n), jnp.bfloat16),
            pltpu.VMEM((4, sub, n), jnp.bfloat16),
            pltpu.VMEM((4, sub, n), jnp.bfloat16),
            pltpu.VMEM((2, sub, n), jnp.bfloat16),
            pltpu.VMEM((2, sub, n), jnp.bfloat16),
            pltpu.SemaphoreType.DMA,
            pltpu.SemaphoreType.DMA((2,)),
            pltpu.SemaphoreType.DMA((2,)),
            pltpu.SemaphoreType.DMA((2,)),
            pltpu.SemaphoreType.DMA((2,)),
            pltpu.SemaphoreType.DMA((2,)),
            pltpu.SemaphoreType.DMA((2,)),
            pltpu.SemaphoreType.DMA((2,)),
            pltpu.SemaphoreType.DMA((2,)),
            pltpu.SemaphoreType.DMA((2,)),
            pltpu.SemaphoreType.DMA((2,)),
            pltpu.SemaphoreType.DMA((2,)),
            pltpu.SemaphoreType.DMA((2,)),
            pltpu.SemaphoreType.DMA((2,)),
            pltpu.SemaphoreType.REGULAR,
            pltpu.SemaphoreType.REGULAR,
            pltpu.SemaphoreType.REGULAR,
            pltpu.SemaphoreType.REGULAR,
        ],
        compiler_params=pltpu.CompilerParams(collective_id=0),
    )(x16)


# device time: 1139874 ns/iter; 1.1012x vs baseline; 1.0001x over previous
import jax
import jax.numpy as jnp
from jax import lax
from jax.experimental import pallas as pl
from jax.experimental.pallas import tpu as pltpu

N_DEV = 4
S = 8
NF = S // 2


def kernel(x):
    x16 = x.astype(jnp.bfloat16)
    m_per, n = x16.shape
    sub = m_per // S

    def body(
        x_ref, out_ref,
        sendbuf, recvL, recvR, recvE, recvF,
        own_sem, load_sems, sendA, sendB, recvC, recvD,
        fwdsR, fwdsL, recvEs, recvFs,
        copyCs, copyDs, copyEs, copyFs,
        credA, credB, credFR, credFL,
    ):
        my = lax.axis_index("i")
        left = (my + N_DEV - 1) % N_DEV
        right = (my + 1) % N_DEV
        opp = (my + 2) % N_DEV
        base_my = my * m_per
        base_l = left * m_per
        base_r = right * m_per
        base_o = opp * m_per

        barrier_sem = pltpu.get_barrier_semaphore()
        for nbr in (left, right):
            pl.semaphore_signal(
                barrier_sem, inc=1,
                device_id=(nbr,), device_id_type=pl.DeviceIdType.MESH,
            )
        pl.semaphore_wait(barrier_sem, 2)

        own_cp = pltpu.make_async_copy(
            x_ref, out_ref.at[pl.ds(base_my, m_per)], own_sem
        )
        own_cp.start()

        loads, A, B, fR, fL = {}, {}, {}, {}, {}
        copyC, copyD, copyE, copyF = {}, {}, {}, {}

        def start_load(k):
            d = pltpu.make_async_copy(
                x_ref.at[pl.ds(k * sub, sub)], sendbuf.at[k % 2],
                load_sems.at[k % 2],
            )
            loads[k] = d
            d.start()

        def f_recv_desc(j):
            return pltpu.make_async_remote_copy(
                src_ref=recvR.at[0], dst_ref=recvF.at[j % 2],
                send_sem=fwdsL.at[j % 2], recv_sem=recvFs.at[j % 2],
                device_id=(left,), device_id_type=pl.DeviceIdType.MESH,
            )

        start_load(0)
        start_load(1)

        for k in range(S):
            if k >= 2 and k % 2 == 0:
                j = (k - 2) // 2
                copyE[j].wait()
                if j <= NF - 3:
                    pl.semaphore_signal(
                        credFR, inc=1,
                        device_id=(left,), device_id_type=pl.DeviceIdType.MESH,
                    )
            if k >= 3 and k % 2 == 1:
                j = (k - 3) // 2
                copyF[j].wait()
                if j <= NF - 3:
                    pl.semaphore_signal(
                        credFL, inc=1,
                        device_id=(right,), device_id_type=pl.DeviceIdType.MESH,
                    )

            loads[k].wait()
            if k >= 4:
                pl.semaphore_wait(credA, 1)
                pl.semaphore_wait(credB, 1)
            A[k] = pltpu.make_async_remote_copy(
                src_ref=sendbuf.at[k % 2], dst_ref=recvL.at[k % 4],
                send_sem=sendA.at[k % 2], recv_sem=recvC.at[k % 2],
                device_id=(right,), device_id_type=pl.DeviceIdType.MESH,
            )
            A[k].start()
            B[k] = pltpu.make_async_remote_copy(
                src_ref=sendbuf.at[k % 2], dst_ref=recvR.at[k % 4],
                send_sem=sendB.at[k % 2], recv_sem=recvD.at[k % 2],
                device_id=(left,), device_id_type=pl.DeviceIdType.MESH,
            )
            B[k].start()

            A[k].wait_recv()
            copyC[k] = pltpu.make_async_copy(
                recvL.at[k % 4],
                out_ref.at[pl.ds(base_l + k * sub, sub)],
                copyCs.at[k % 2],
            )
            copyC[k].start()
            if k % 2 == 0:
                j = k // 2
                if j >= 2:
                    pl.semaphore_wait(credFR, 1)
                fR[j] = pltpu.make_async_remote_copy(
                    src_ref=recvL.at[k % 4], dst_ref=recvE.at[j % 2],
                    send_sem=fwdsR.at[j % 2], recv_sem=recvEs.at[j % 2],
                    device_id=(right,), device_id_type=pl.DeviceIdType.MESH,
                )
                fR[j].start()

            B[k].wait_recv()
            copyD[k] = pltpu.make_async_copy(
                recvR.at[k % 4],
                out_ref.at[pl.ds(base_r + k * sub, sub)],
                copyDs.at[k % 2],
            )
            copyD[k].start()
            if k % 2 == 1:
                j = (k - 1) // 2
                if j >= 2:
                    pl.semaphore_wait(credFL, 1)
                fL[j] = pltpu.make_async_remote_copy(
                    src_ref=recvR.at[k % 4], dst_ref=recvF.at[j % 2],
                    send_sem=fwdsL.at[j % 2], recv_sem=recvFs.at[j % 2],
                    device_id=(left,), device_id_type=pl.DeviceIdType.MESH,
                )
                fL[j].start()

            if k >= 1:
                kk = k - 1
                copyC[kk].wait()
                if kk % 2 == 0:
                    fR[kk // 2].wait_send()
                if kk <= S - 5:
                    pl.semaphore_signal(
                        credA, inc=1,
                        device_id=(left,), device_id_type=pl.DeviceIdType.MESH,
                    )
                copyD[kk].wait()
                if kk % 2 == 1:
                    fL[(kk - 1) // 2].wait_send()
                if kk <= S - 5:
                    pl.semaphore_signal(
                        credB, inc=1,
                        device_id=(right,), device_id_type=pl.DeviceIdType.MESH,
                    )

            if k % 2 == 1:
                j = (k - 1) // 2
                erecv = pltpu.make_async_remote_copy(
                    src_ref=recvL.at[0], dst_ref=recvE.at[j % 2],
                    send_sem=fwdsR.at[j % 2], recv_sem=recvEs.at[j % 2],
                    device_id=(right,), device_id_type=pl.DeviceIdType.MESH,
                )
                erecv.wait_recv()
                copyE[j] = pltpu.make_async_copy(
                    recvE.at[j % 2],
                    out_ref.at[pl.ds(base_o + 2 * j * sub, sub)],
                    copyEs.at[j % 2],
                )
                copyE[j].start()

            if k >= 2 and k % 2 == 0:
                j = (k - 2) // 2
                f_recv_desc(j).wait_recv()
                copyF[j] = pltpu.make_async_copy(
                    recvF.at[j % 2],
                    out_ref.at[pl.ds(base_o + (2 * j + 1) * sub, sub)],
                    copyFs.at[j % 2],
                )
                copyF[j].start()

            if k + 2 < S:
                A[k].wait_send()
                B[k].wait_send()
                start_load(k + 2)

        for k in (S - 2, S - 1):
            A[k].wait_send()
            B[k].wait_send()
        kk = S - 1
        copyC[kk].wait()
        copyD[kk].wait()
        fL[(kk - 1) // 2].wait_send()
        copyE[NF - 1].wait()
        j = NF - 1
        f_recv_desc(j).wait_recv()
        lastF = pltpu.make_async_copy(
            recvF.at[j % 2],
            out_ref.at[pl.ds(base_o + (2 * j + 1) * sub, sub)],
            copyFs.at[j % 2],
        )
        lastF.start()
        lastF.wait()
        own_cp.wait()

    return pl.pallas_call(
        body,
        out_shape=jax.ShapeDtypeStruct((N_DEV * m_per, n), jnp.bfloat16),
        in_specs=[pl.BlockSpec(memory_space=pl.ANY)],
        out_specs=pl.BlockSpec(memory_space=pl.ANY),
        scratch_shapes=[
            pltpu.VMEM((2, sub, n), jnp.bfloat16),
            pltpu.VMEM((4, sub, n), jnp.bfloat16),
            pltpu.VMEM((4, sub, n), jnp.bfloat16),
            pltpu.VMEM((2, sub, n), jnp.bfloat16),
            pltpu.VMEM((2, sub, n), jnp.bfloat16),
            pltpu.SemaphoreType.DMA,
            pltpu.SemaphoreType.DMA((2,)),
            pltpu.SemaphoreType.DMA((2,)),
            pltpu.SemaphoreType.DMA((2,)),
            pltpu.SemaphoreType.DMA((2,)),
            pltpu.SemaphoreType.DMA((2,)),
            pltpu.SemaphoreType.DMA((2,)),
            pltpu.SemaphoreType.DMA((2,)),
            pltpu.SemaphoreType.DMA((2,)),
            pltpu.SemaphoreType.DMA((2,)),
            pltpu.SemaphoreType.DMA((2,)),
            pltpu.SemaphoreType.DMA((2,)),
            pltpu.SemaphoreType.DMA((2,)),
            pltpu.SemaphoreType.DMA((2,)),
            pltpu.SemaphoreType.REGULAR,
            pltpu.SemaphoreType.REGULAR,
            pltpu.SemaphoreType.REGULAR,
            pltpu.SemaphoreType.REGULAR,
        ],
        compiler_params=pltpu.CompilerParams(
            collective_id=0, vmem_limit_bytes=60 * 1024 * 1024
        ),
    )(x16)


# device time: 740276 ns/iter; 1.6957x vs baseline; 1.5398x over previous
import jax
import jax.numpy as jnp
from jax import lax
from jax.experimental import pallas as pl
from jax.experimental.pallas import tpu as pltpu

N_DEV = 4
S = 8
NF = S // 2


def kernel(x):
    x16 = x.astype(jnp.bfloat16)
    m_per, n = x16.shape
    sub = m_per // S

    def body(
        x_ref, out_ref,
        sendbuf, recvL, recvR, recvE, recvF,
        copyOs, load_sems, sendA, sendB, recvC, recvD,
        fwdsR, fwdsL, recvEs, recvFs,
        copyCs, copyDs, copyEs, copyFs,
        credA, credB, credFR, credFL,
    ):
        my = lax.axis_index("i")
        left = (my + N_DEV - 1) % N_DEV
        right = (my + 1) % N_DEV
        opp = (my + 2) % N_DEV
        base_my = my * m_per
        base_l = left * m_per
        base_r = right * m_per
        base_o = opp * m_per

        barrier_sem = pltpu.get_barrier_semaphore()
        for nbr in (left, right):
            pl.semaphore_signal(
                barrier_sem, inc=1,
                device_id=(nbr,), device_id_type=pl.DeviceIdType.MESH,
            )
        pl.semaphore_wait(barrier_sem, 2)

        loads, A, B, fR, fL = {}, {}, {}, {}, {}
        copyC, copyD, copyE, copyF, copyO = {}, {}, {}, {}, {}

        def start_load(k):
            d = pltpu.make_async_copy(
                x_ref.at[pl.ds(k * sub, sub)], sendbuf.at[k % 2],
                load_sems.at[k % 2],
            )
            loads[k] = d
            d.start()

        def f_recv_desc(j):
            return pltpu.make_async_remote_copy(
                src_ref=recvR.at[0], dst_ref=recvF.at[j % 2],
                send_sem=fwdsL.at[j % 2], recv_sem=recvFs.at[j % 2],
                device_id=(left,), device_id_type=pl.DeviceIdType.MESH,
            )

        start_load(0)
        start_load(1)

        for k in range(S):
            if k >= 2 and k % 2 == 0:
                j = (k - 2) // 2
                copyE[j].wait()
                if j <= NF - 3:
                    pl.semaphore_signal(
                        credFR, inc=1,
                        device_id=(left,), device_id_type=pl.DeviceIdType.MESH,
                    )
            if k >= 3 and k % 2 == 1:
                j = (k - 3) // 2
                copyF[j].wait()
                if j <= NF - 3:
                    pl.semaphore_signal(
                        credFL, inc=1,
                        device_id=(right,), device_id_type=pl.DeviceIdType.MESH,
                    )

            loads[k].wait()
            if k >= 4:
                pl.semaphore_wait(credA, 1)
                pl.semaphore_wait(credB, 1)
            A[k] = pltpu.make_async_remote_copy(
                src_ref=sendbuf.at[k % 2], dst_ref=recvL.at[k % 4],
                send_sem=sendA.at[k % 2], recv_sem=recvC.at[k % 2],
                device_id=(right,), device_id_type=pl.DeviceIdType.MESH,
            )
            A[k].start()
            B[k] = pltpu.make_async_remote_copy(
                src_ref=sendbuf.at[k % 2], dst_ref=recvR.at[k % 4],
                send_sem=sendB.at[k % 2], recv_sem=recvD.at[k % 2],
                device_id=(left,), device_id_type=pl.DeviceIdType.MESH,
            )
            B[k].start()
            copyO[k] = pltpu.make_async_copy(
                sendbuf.at[k % 2],
                out_ref.at[pl.ds(base_my + k * sub, sub)],
                copyOs.at[k % 2],
            )
            copyO[k].start()

            A[k].wait_recv()
            copyC[k] = pltpu.make_async_copy(
                recvL.at[k % 4],
                out_ref.at[pl.ds(base_l + k * sub, sub)],
                copyCs.at[k % 2],
            )
            copyC[k].start()
            if k % 2 == 0:
                j = k // 2
                if j >= 2:
                    pl.semaphore_wait(credFR, 1)
                fR[j] = pltpu.make_async_remote_copy(
                    src_ref=recvL.at[k % 4], dst_ref=recvE.at[j % 2],
                    send_sem=fwdsR.at[j % 2], recv_sem=recvEs.at[j % 2],
                    device_id=(right,), device_id_type=pl.DeviceIdType.MESH,
                )
                fR[j].start()

            B[k].wait_recv()
            copyD[k] = pltpu.make_async_copy(
                recvR.at[k % 4],
                out_ref.at[pl.ds(base_r + k * sub, sub)],
                copyDs.at[k % 2],
            )
            copyD[k].start()
            if k % 2 == 1:
                j = (k - 1) // 2
                if j >= 2:
                    pl.semaphore_wait(credFL, 1)
                fL[j] = pltpu.make_async_remote_copy(
                    src_ref=recvR.at[k % 4], dst_ref=recvF.at[j % 2],
                    send_sem=fwdsL.at[j % 2], recv_sem=recvFs.at[j % 2],
                    device_id=(left,), device_id_type=pl.DeviceIdType.MESH,
                )
                fL[j].start()

            if k >= 1:
                kk = k - 1
                copyC[kk].wait()
                if kk % 2 == 0:
                    fR[kk // 2].wait_send()
                if kk <= S - 5:
                    pl.semaphore_signal(
                        credA, inc=1,
                        device_id=(left,), device_id_type=pl.DeviceIdType.MESH,
                    )
                copyD[kk].wait()
                if kk % 2 == 1:
                    fL[(kk - 1) // 2].wait_send()
                if kk <= S - 5:
                    pl.semaphore_signal(
                        credB, inc=1,
                        device_id=(right,), device_id_type=pl.DeviceIdType.MESH,
                    )

            if k % 2 == 1:
                j = (k - 1) // 2
                erecv = pltpu.make_async_remote_copy(
                    src_ref=recvL.at[0], dst_ref=recvE.at[j % 2],
                    send_sem=fwdsR.at[j % 2], recv_sem=recvEs.at[j % 2],
                    device_id=(right,), device_id_type=pl.DeviceIdType.MESH,
                )
                erecv.wait_recv()
                copyE[j] = pltpu.make_async_copy(
                    recvE.at[j % 2],
                    out_ref.at[pl.ds(base_o + 2 * j * sub, sub)],
                    copyEs.at[j % 2],
                )
                copyE[j].start()

            if k >= 2 and k % 2 == 0:
                j = (k - 2) // 2
                f_recv_desc(j).wait_recv()
                copyF[j] = pltpu.make_async_copy(
                    recvF.at[j % 2],
                    out_ref.at[pl.ds(base_o + (2 * j + 1) * sub, sub)],
                    copyFs.at[j % 2],
                )
                copyF[j].start()

            if k + 2 < S:
                A[k].wait_send()
                B[k].wait_send()
                copyO[k].wait()
                start_load(k + 2)

        for k in (S - 2, S - 1):
            A[k].wait_send()
            B[k].wait_send()
            copyO[k].wait()
        kk = S - 1
        copyC[kk].wait()
        copyD[kk].wait()
        fL[(kk - 1) // 2].wait_send()
        copyE[NF - 1].wait()
        j = NF - 1
        f_recv_desc(j).wait_recv()
        lastF = pltpu.make_async_copy(
            recvF.at[j % 2],
            out_ref.at[pl.ds(base_o + (2 * j + 1) * sub, sub)],
            copyFs.at[j % 2],
        )
        lastF.start()
        lastF.wait()

    return pl.pallas_call(
        body,
        out_shape=jax.ShapeDtypeStruct((N_DEV * m_per, n), jnp.bfloat16),
        in_specs=[pl.BlockSpec(memory_space=pl.ANY)],
        out_specs=pl.BlockSpec(memory_space=pl.ANY),
        scratch_shapes=[
            pltpu.VMEM((2, sub, n), jnp.bfloat16),
            pltpu.VMEM((4, sub, n), jnp.bfloat16),
            pltpu.VMEM((4, sub, n), jnp.bfloat16),
            pltpu.VMEM((2, sub, n), jnp.bfloat16),
            pltpu.VMEM((2, sub, n), jnp.bfloat16),
            pltpu.SemaphoreType.DMA((2,)),
            pltpu.SemaphoreType.DMA((2,)),
            pltpu.SemaphoreType.DMA((2,)),
            pltpu.SemaphoreType.DMA((2,)),
            pltpu.SemaphoreType.DMA((2,)),
            pltpu.SemaphoreType.DMA((2,)),
            pltpu.SemaphoreType.DMA((2,)),
            pltpu.SemaphoreType.DMA((2,)),
            pltpu.SemaphoreType.DMA((2,)),
            pltpu.SemaphoreType.DMA((2,)),
            pltpu.SemaphoreType.DMA((2,)),
            pltpu.SemaphoreType.DMA((2,)),
            pltpu.SemaphoreType.DMA((2,)),
            pltpu.SemaphoreType.DMA((2,)),
            pltpu.SemaphoreType.REGULAR,
            pltpu.SemaphoreType.REGULAR,
            pltpu.SemaphoreType.REGULAR,
            pltpu.SemaphoreType.REGULAR,
        ],
        compiler_params=pltpu.CompilerParams(
            collective_id=0, vmem_limit_bytes=60 * 1024 * 1024
        ),
    )(x16)
